# baseline (speedup 1.0000x reference)
import functools

import jax
import jax.numpy as jnp
from jax import lax
from jax.experimental import pallas as pl
from jax.experimental.pallas import tpu as pltpu
from jax.experimental.pallas import tpu_sc as plsc

N_NODES = 10000
HIDDEN = 128
N_EDGES = 320000
LN_EPS = 1e-5

NC = 2
NS = 16
NW = NC * NS
EH = N_EDGES // 2
EPW = EH // NW
GCHUNK = 40
GRING = 2
NCHG = EPW // GCHUNK
SCHUNK = 40
SRING = 4
NCHS = EPW // SCHUNK
NPS = 632
NPS_LAST = N_NODES - NPS * (NS - 1)

_F32 = jnp.float32


def _gather_body(ebase, x_hbm, src_hbm, dst_hbm, srcg_hbm, dstg_hbm,
                 xs, idx_s, idx_d, rows_s, rows_d, sem_ix, sem_g,
                 sem_ws, sem_wd):
    c = lax.axis_index("c")
    s = lax.axis_index("s")
    wid = s * NC + c
    base_out = wid * EPW
    base_in = ebase + base_out

    row0 = pl.multiple_of(s * NPS, 8)

    @pl.when(s < NS - 1)
    def _():
        pltpu.sync_copy(x_hbm.at[pl.ds(row0, NPS)], xs.at[pl.ds(row0, NPS)])

    @pl.when(s == NS - 1)
    def _():
        pltpu.sync_copy(x_hbm.at[pl.ds(NPS * (NS - 1), NPS_LAST)],
                        xs.at[pl.ds(NPS * (NS - 1), NPS_LAST)])

    plsc.subcore_barrier()

    def issue_idx(i, b):
        off = pl.multiple_of(base_in + i * GCHUNK, 8)
        pltpu.async_copy(src_hbm.at[pl.ds(off, GCHUNK)], idx_s[b], sem_ix[b])
        pltpu.async_copy(dst_hbm.at[pl.ds(off, GCHUNK)], idx_d[b], sem_ix[b])

    def wait_idx(b):
        pltpu.make_async_copy(src_hbm.at[pl.ds(0, GCHUNK)], idx_s[b],
                              sem_ix[b]).wait()
        pltpu.make_async_copy(dst_hbm.at[pl.ds(0, GCHUNK)], idx_d[b],
                              sem_ix[b]).wait()

    def wait_writes(b):
        pltpu.make_async_copy(
            rows_s[b], srcg_hbm.at[pl.ds(0, GCHUNK)], sem_ws[b]).wait()
        pltpu.make_async_copy(
            rows_d[b], dstg_hbm.at[pl.ds(0, GCHUNK)], sem_wd[b]).wait()

    def step(i, b, wait_w, last):
        off = pl.multiple_of(base_out + i * GCHUNK, 8)
        if wait_w:
            wait_writes(b)
        wait_idx(b)
        cp_s = pltpu.async_copy(xs.at[idx_s[b]], rows_s[b], sem_g)
        cp_d = pltpu.async_copy(xs.at[idx_d[b]], rows_d[b], sem_g)
        cp_s.wait()
        cp_d.wait()
        if not last:
            if isinstance(i, int):
                if i + GRING < NCHG:
                    issue_idx(i + GRING, b)
            else:
                @pl.when(i + GRING < NCHG)
                def _():
                    issue_idx(i + GRING, b)
        pltpu.async_copy(rows_s[b], srcg_hbm.at[pl.ds(off, GCHUNK)],
                         sem_ws[b])
        pltpu.async_copy(rows_d[b], dstg_hbm.at[pl.ds(off, GCHUNK)],
                         sem_wd[b])

    for b in range(GRING):
        issue_idx(b, b)

    for b in range(GRING):
        step(b, b, wait_w=False, last=False)

    @pl.loop(1, NCHG // GRING)
    def _pair(j):
        for b in range(GRING):
            step(j * GRING + b, b, wait_w=True, last=False)

    for t in range(NCHG - (NCHG // GRING) * GRING):
        i = (NCHG // GRING) * GRING + t
        step(i, i % GRING, wait_w=True, last=True)

    for b in range(GRING):
        wait_writes(b)


@functools.cache
def _make_gather(ebase):
    return pl.kernel(
        functools.partial(_gather_body, ebase),
        out_type=(
            jax.ShapeDtypeStruct((EH, HIDDEN), _F32),
            jax.ShapeDtypeStruct((EH, HIDDEN), _F32),
        ),
        mesh=plsc.VectorSubcoreMesh(core_axis_name="c", subcore_axis_name="s"),
        scratch_types=[
            pltpu.VMEM_SHARED((N_NODES, HIDDEN), _F32),
            [pltpu.VMEM((GCHUNK,), jnp.int32) for _ in range(GRING)],
            [pltpu.VMEM((GCHUNK,), jnp.int32) for _ in range(GRING)],
            [pltpu.VMEM((GCHUNK, HIDDEN), _F32) for _ in range(GRING)],
            [pltpu.VMEM((GCHUNK, HIDDEN), _F32) for _ in range(GRING)],
            [pltpu.SemaphoreType.DMA for _ in range(GRING)],
            pltpu.SemaphoreType.DMA,
            [pltpu.SemaphoreType.DMA for _ in range(GRING)],
            [pltpu.SemaphoreType.DMA for _ in range(GRING)],
        ],
    )


def _scatter_body(ebase, msg_hbm, dsti_hbm, zeros_hbm, out_hbm, idx_v, msg_v,
                  sem_ld, acc):
    c = lax.axis_index("c")
    s = lax.axis_index("s")
    wid = s * NC + c
    base_msg = wid * EPW
    base_in = ebase + base_msg

    row0 = pl.multiple_of(s * NPS, 8)

    @pl.when(s < NS - 1)
    def _():
        pltpu.sync_copy(zeros_hbm.at[pl.ds(row0, NPS)],
                        acc.at[pl.ds(row0, NPS)])

    @pl.when(s == NS - 1)
    def _():
        pltpu.sync_copy(zeros_hbm.at[pl.ds(NPS * (NS - 1), NPS_LAST)],
                        acc.at[pl.ds(NPS * (NS - 1), NPS_LAST)])

    plsc.subcore_barrier()

    def issue_load(i, b):
        offi = pl.multiple_of(base_in + i * SCHUNK, 8)
        offm = pl.multiple_of(base_msg + i * SCHUNK, 8)
        pltpu.async_copy(dsti_hbm.at[pl.ds(offi, SCHUNK)], idx_v[b],
                         sem_ld[b])
        pltpu.async_copy(msg_hbm.at[pl.ds(offm, SCHUNK)], msg_v[b],
                         sem_ld[b])

    def wait_load(b):
        pltpu.make_async_copy(dsti_hbm.at[pl.ds(0, SCHUNK)], idx_v[b],
                              sem_ld[b]).wait()
        pltpu.make_async_copy(msg_hbm.at[pl.ds(0, SCHUNK)], msg_v[b],
                              sem_ld[b]).wait()

    def sstep(i, b, last):
        wait_load(b)
        pltpu.sync_copy(msg_v[b], acc.at[idx_v[b]], add=True)
        if not last:
            if isinstance(i, int):
                if i + SRING < NCHS:
                    issue_load(i + SRING, b)
            else:
                @pl.when(i + SRING < NCHS)
                def _():
                    issue_load(i + SRING, b)

    for b in range(SRING):
        issue_load(b, b)
    for b in range(SRING):
        sstep(b, b, last=False)

    @pl.loop(1, NCHS // SRING)
    def _ring(j):
        for b in range(SRING):
            sstep(j * SRING + b, b, last=False)

    for t in range(NCHS - (NCHS // SRING) * SRING):
        i = (NCHS // SRING) * SRING + t
        sstep(i, i % SRING, last=True)

    plsc.subcore_barrier()

    @pl.when(s < NS - 1)
    def _():
        pltpu.sync_copy(acc.at[pl.ds(row0, NPS)],
                        out_hbm.at[c].at[pl.ds(row0, NPS)])

    @pl.when(s == NS - 1)
    def _():
        pltpu.sync_copy(acc.at[pl.ds(NPS * (NS - 1), NPS_LAST)],
                        out_hbm.at[c].at[pl.ds(NPS * (NS - 1), NPS_LAST)])


@functools.cache
def _make_scatter(ebase):
    return pl.kernel(
        functools.partial(_scatter_body, ebase),
        out_type=jax.ShapeDtypeStruct((NC, N_NODES, HIDDEN), _F32),
        mesh=plsc.VectorSubcoreMesh(core_axis_name="c", subcore_axis_name="s"),
        scratch_types=[
            [pltpu.VMEM((SCHUNK,), jnp.int32) for _ in range(SRING)],
            [pltpu.VMEM((SCHUNK, HIDDEN), _F32) for _ in range(SRING)],
            [pltpu.SemaphoreType.DMA for _ in range(SRING)],
            pltpu.VMEM_SHARED((N_NODES, HIDDEN), _F32),
        ],
    )


BE = 2000


def _edge_mlp_body(src_ref, dst_ref, w_ref, Ws_ref, Wd_ref, wrow_ref,
                   b1_ref, W2_ref, b2_ref, out_ref):
    w = w_ref[...]
    z = jnp.dot(src_ref[...].astype(jnp.bfloat16), Ws_ref[...],
                preferred_element_type=_F32)
    z += jnp.dot(dst_ref[...].astype(jnp.bfloat16), Wd_ref[...],
                 preferred_element_type=_F32)
    z += w * wrow_ref[...] + b1_ref[...]
    h = z * jax.nn.sigmoid(z)
    m = jnp.dot(h.astype(jnp.bfloat16), W2_ref[...],
                preferred_element_type=_F32) + b2_ref[...]
    out_ref[...] = m * w


def _full(shape):
    return pl.BlockSpec(shape, lambda i: (0, 0))


_edge_mlp = pl.pallas_call(
    _edge_mlp_body,
    grid=(EH // BE,),
    in_specs=[
        pl.BlockSpec((BE, HIDDEN), lambda i: (i, 0)),
        pl.BlockSpec((BE, HIDDEN), lambda i: (i, 0)),
        pl.BlockSpec((BE, 1), lambda i: (i, 0)),
        _full((HIDDEN, HIDDEN)),
        _full((HIDDEN, HIDDEN)),
        _full((1, HIDDEN)),
        _full((1, HIDDEN)),
        _full((HIDDEN, HIDDEN)),
        _full((1, HIDDEN)),
    ],
    out_specs=pl.BlockSpec((BE, HIDDEN), lambda i: (i, 0)),
    out_shape=jax.ShapeDtypeStruct((EH, HIDDEN), _F32),
)


RB = 2000


def _update_body(x_ref, a0_ref, a1_ref, a2_ref, a3_ref, W1x_ref, W1a_ref,
                 b1_ref, g_ref, bln_ref, W2_ref, b2_ref, out_ref):
    xb = x_ref[...]
    agg = (a0_ref[...] + a1_ref[...]) + (a2_ref[...] + a3_ref[...])
    u = jnp.dot(xb, W1x_ref[...], preferred_element_type=_F32,
                precision=lax.Precision.HIGHEST)
    u += jnp.dot(agg, W1a_ref[...], preferred_element_type=_F32,
                 precision=lax.Precision.HIGHEST)
    u += b1_ref[...]
    mu = jnp.mean(u, axis=-1, keepdims=True)
    var = jnp.mean((u - mu) * (u - mu), axis=-1, keepdims=True)
    un = (u - mu) * lax.rsqrt(var + LN_EPS) * g_ref[...] + bln_ref[...]
    h = un * jax.nn.sigmoid(un)
    out_ref[...] = (jnp.dot(h, W2_ref[...], preferred_element_type=_F32,
                            precision=lax.Precision.HIGHEST)
                    + b2_ref[...] + xb)


_update = pl.pallas_call(
    _update_body,
    grid=(N_NODES // RB,),
    in_specs=[
        pl.BlockSpec((RB, HIDDEN), lambda i: (i, 0)),
        pl.BlockSpec((RB, HIDDEN), lambda i: (i, 0)),
        pl.BlockSpec((RB, HIDDEN), lambda i: (i, 0)),
        pl.BlockSpec((RB, HIDDEN), lambda i: (i, 0)),
        pl.BlockSpec((RB, HIDDEN), lambda i: (i, 0)),
        _full((HIDDEN, HIDDEN)),
        _full((HIDDEN, HIDDEN)),
        _full((1, HIDDEN)),
        _full((1, HIDDEN)),
        _full((1, HIDDEN)),
        _full((HIDDEN, HIDDEN)),
        _full((1, HIDDEN)),
    ],
    out_specs=pl.BlockSpec((RB, HIDDEN), lambda i: (i, 0)),
    out_shape=jax.ShapeDtypeStruct((N_NODES, HIDDEN), _F32),
)


def kernel(x, edge_index, edge_weight, W1m, b1m, W2m, b2m, W1u, b1u,
           ln_g, ln_b, W2u, b2u):
    src = edge_index[0].astype(jnp.int32)
    dst = edge_index[1].astype(jnp.int32)

    Ws = W1m[:HIDDEN].astype(jnp.bfloat16)
    Wd = W1m[HIDDEN:2 * HIDDEN].astype(jnp.bfloat16)
    wrow = W1m[2 * HIDDEN:].reshape(1, -1)
    b1 = b1m.reshape(1, -1)
    W2 = W2m.astype(jnp.bfloat16)
    b2 = b2m.reshape(1, -1)
    zeros = jnp.zeros((N_NODES, HIDDEN), _F32)
    w2d = edge_weight.reshape(-1, 1)

    halves = []
    for h in range(2):
        ebase = h * EH
        src_g, dst_g = _make_gather(ebase)(x, src, dst)
        msg = _edge_mlp(src_g, dst_g, w2d[ebase:ebase + EH],
                        Ws, Wd, wrow, b1, W2, b2)
        parts = _make_scatter(ebase)(msg, dst, zeros)
        halves.append(parts)

    pa, pb = halves
    out = _update(
        x, pa[0], pa[1], pb[0], pb[1],
        W1u[:HIDDEN], W1u[HIDDEN:], b1u.reshape(1, -1),
        ln_g.reshape(1, -1), ln_b.reshape(1, -1), W2u, b2u.reshape(1, -1),
    )
    return out


# speedup vs baseline: 1.2983x; 1.2983x over previous
import functools

import jax
import jax.numpy as jnp
from jax import lax
from jax.experimental import pallas as pl
from jax.experimental.pallas import tpu as pltpu
from jax.experimental.pallas import tpu_sc as plsc

N_NODES = 10000
HIDDEN = 128
N_EDGES = 320000
LN_EPS = 1e-5

NC = 2
NS = 16
NW = NC * NS
EH = N_EDGES
EPW = EH // NW
GCHUNK = 80
GRING = 2
NCHG = EPW // GCHUNK
SCHUNK = 80
SRING = 4
NCHS = EPW // SCHUNK
NPS = 632
NPS_LAST = N_NODES - NPS * (NS - 1)

_F32 = jnp.float32


def _gather_body(ebase, x_hbm, src_hbm, dst_hbm, srcg_hbm, dstg_hbm,
                 xs, idx_s, idx_d, rows_s, rows_d, sem_ix, sem_g,
                 sem_ws, sem_wd):
    c = lax.axis_index("c")
    s = lax.axis_index("s")
    wid = s * NC + c
    base_out = wid * EPW
    base_in = ebase + base_out

    row0 = pl.multiple_of(s * NPS, 8)

    @pl.when(s < NS - 1)
    def _():
        pltpu.sync_copy(x_hbm.at[pl.ds(row0, NPS)], xs.at[pl.ds(row0, NPS)])

    @pl.when(s == NS - 1)
    def _():
        pltpu.sync_copy(x_hbm.at[pl.ds(NPS * (NS - 1), NPS_LAST)],
                        xs.at[pl.ds(NPS * (NS - 1), NPS_LAST)])

    plsc.subcore_barrier()

    def issue_idx(i, b):
        off = pl.multiple_of(base_in + i * GCHUNK, 8)
        pltpu.async_copy(src_hbm.at[pl.ds(off, GCHUNK)], idx_s[b], sem_ix[b])
        pltpu.async_copy(dst_hbm.at[pl.ds(off, GCHUNK)], idx_d[b], sem_ix[b])

    def wait_idx(b):
        pltpu.make_async_copy(src_hbm.at[pl.ds(0, GCHUNK)], idx_s[b],
                              sem_ix[b]).wait()
        pltpu.make_async_copy(dst_hbm.at[pl.ds(0, GCHUNK)], idx_d[b],
                              sem_ix[b]).wait()

    def wait_writes(b):
        pltpu.make_async_copy(
            rows_s[b], srcg_hbm.at[pl.ds(0, GCHUNK)], sem_ws[b]).wait()
        pltpu.make_async_copy(
            rows_d[b], dstg_hbm.at[pl.ds(0, GCHUNK)], sem_wd[b]).wait()

    def step(i, b, wait_w, last):
        off = pl.multiple_of(base_out + i * GCHUNK, 8)
        if wait_w:
            wait_writes(b)
        wait_idx(b)
        cp_s = pltpu.async_copy(xs.at[idx_s[b]], rows_s[b], sem_g)
        cp_d = pltpu.async_copy(xs.at[idx_d[b]], rows_d[b], sem_g)
        cp_s.wait()
        cp_d.wait()
        if not last:
            if isinstance(i, int):
                if i + GRING < NCHG:
                    issue_idx(i + GRING, b)
            else:
                @pl.when(i + GRING < NCHG)
                def _():
                    issue_idx(i + GRING, b)
        pltpu.async_copy(rows_s[b], srcg_hbm.at[pl.ds(off, GCHUNK)],
                         sem_ws[b])
        pltpu.async_copy(rows_d[b], dstg_hbm.at[pl.ds(off, GCHUNK)],
                         sem_wd[b])

    for b in range(GRING):
        issue_idx(b, b)

    for b in range(GRING):
        step(b, b, wait_w=False, last=False)

    @pl.loop(1, NCHG // GRING)
    def _pair(j):
        for b in range(GRING):
            step(j * GRING + b, b, wait_w=True, last=False)

    for t in range(NCHG - (NCHG // GRING) * GRING):
        i = (NCHG // GRING) * GRING + t
        step(i, i % GRING, wait_w=True, last=True)

    for b in range(GRING):
        wait_writes(b)


@functools.cache
def _make_gather(ebase):
    return pl.kernel(
        functools.partial(_gather_body, ebase),
        out_type=(
            jax.ShapeDtypeStruct((EH, HIDDEN), _F32),
            jax.ShapeDtypeStruct((EH, HIDDEN), _F32),
        ),
        mesh=plsc.VectorSubcoreMesh(core_axis_name="c", subcore_axis_name="s"),
        scratch_types=[
            pltpu.VMEM_SHARED((N_NODES, HIDDEN), _F32),
            [pltpu.VMEM((GCHUNK,), jnp.int32) for _ in range(GRING)],
            [pltpu.VMEM((GCHUNK,), jnp.int32) for _ in range(GRING)],
            [pltpu.VMEM((GCHUNK, HIDDEN), _F32) for _ in range(GRING)],
            [pltpu.VMEM((GCHUNK, HIDDEN), _F32) for _ in range(GRING)],
            [pltpu.SemaphoreType.DMA for _ in range(GRING)],
            pltpu.SemaphoreType.DMA,
            [pltpu.SemaphoreType.DMA for _ in range(GRING)],
            [pltpu.SemaphoreType.DMA for _ in range(GRING)],
        ],
    )


def _scatter_body(ebase, msg_hbm, dsti_hbm, zeros_hbm, out_hbm, idx_v, msg_v,
                  sem_ld, acc):
    c = lax.axis_index("c")
    s = lax.axis_index("s")
    wid = s * NC + c
    base_msg = wid * EPW
    base_in = ebase + base_msg

    row0 = pl.multiple_of(s * NPS, 8)

    @pl.when(s < NS - 1)
    def _():
        pltpu.sync_copy(zeros_hbm.at[pl.ds(row0, NPS)],
                        acc.at[pl.ds(row0, NPS)])

    @pl.when(s == NS - 1)
    def _():
        pltpu.sync_copy(zeros_hbm.at[pl.ds(NPS * (NS - 1), NPS_LAST)],
                        acc.at[pl.ds(NPS * (NS - 1), NPS_LAST)])

    plsc.subcore_barrier()

    def issue_load(i, b):
        offi = pl.multiple_of(base_in + i * SCHUNK, 8)
        offm = pl.multiple_of(base_msg + i * SCHUNK, 8)
        pltpu.async_copy(dsti_hbm.at[pl.ds(offi, SCHUNK)], idx_v[b],
                         sem_ld[b])
        pltpu.async_copy(msg_hbm.at[pl.ds(offm, SCHUNK)], msg_v[b],
                         sem_ld[b])

    def wait_load(b):
        pltpu.make_async_copy(dsti_hbm.at[pl.ds(0, SCHUNK)], idx_v[b],
                              sem_ld[b]).wait()
        pltpu.make_async_copy(msg_hbm.at[pl.ds(0, SCHUNK)], msg_v[b],
                              sem_ld[b]).wait()

    def sstep(i, b, last):
        wait_load(b)
        pltpu.sync_copy(msg_v[b], acc.at[idx_v[b]], add=True)
        if not last:
            if isinstance(i, int):
                if i + SRING < NCHS:
                    issue_load(i + SRING, b)
            else:
                @pl.when(i + SRING < NCHS)
                def _():
                    issue_load(i + SRING, b)

    for b in range(SRING):
        issue_load(b, b)
    for b in range(SRING):
        sstep(b, b, last=False)

    @pl.loop(1, NCHS // SRING)
    def _ring(j):
        for b in range(SRING):
            sstep(j * SRING + b, b, last=False)

    for t in range(NCHS - (NCHS // SRING) * SRING):
        i = (NCHS // SRING) * SRING + t
        sstep(i, i % SRING, last=True)

    plsc.subcore_barrier()

    @pl.when(s < NS - 1)
    def _():
        pltpu.sync_copy(acc.at[pl.ds(row0, NPS)],
                        out_hbm.at[c].at[pl.ds(row0, NPS)])

    @pl.when(s == NS - 1)
    def _():
        pltpu.sync_copy(acc.at[pl.ds(NPS * (NS - 1), NPS_LAST)],
                        out_hbm.at[c].at[pl.ds(NPS * (NS - 1), NPS_LAST)])


@functools.cache
def _make_scatter(ebase):
    return pl.kernel(
        functools.partial(_scatter_body, ebase),
        out_type=jax.ShapeDtypeStruct((NC, N_NODES, HIDDEN), _F32),
        mesh=plsc.VectorSubcoreMesh(core_axis_name="c", subcore_axis_name="s"),
        scratch_types=[
            [pltpu.VMEM((SCHUNK,), jnp.int32) for _ in range(SRING)],
            [pltpu.VMEM((SCHUNK, HIDDEN), _F32) for _ in range(SRING)],
            [pltpu.SemaphoreType.DMA for _ in range(SRING)],
            pltpu.VMEM_SHARED((N_NODES, HIDDEN), _F32),
        ],
    )


BE = 4000


def _edge_mlp_body(src_ref, dst_ref, w_ref, Ws_ref, Wd_ref, wrow_ref,
                   b1_ref, W2_ref, b2_ref, out_ref):
    w = w_ref[...]
    z = jnp.dot(src_ref[...].astype(jnp.bfloat16), Ws_ref[...],
                preferred_element_type=_F32)
    z += jnp.dot(dst_ref[...].astype(jnp.bfloat16), Wd_ref[...],
                 preferred_element_type=_F32)
    z += w * wrow_ref[...] + b1_ref[...]
    h = z * jax.nn.sigmoid(z)
    m = jnp.dot(h.astype(jnp.bfloat16), W2_ref[...],
                preferred_element_type=_F32) + b2_ref[...]
    out_ref[...] = m * w


def _full(shape):
    return pl.BlockSpec(shape, lambda i: (0, 0))


_edge_mlp = pl.pallas_call(
    _edge_mlp_body,
    grid=(EH // BE,),
    in_specs=[
        pl.BlockSpec((BE, HIDDEN), lambda i: (i, 0)),
        pl.BlockSpec((BE, HIDDEN), lambda i: (i, 0)),
        pl.BlockSpec((BE, 1), lambda i: (i, 0)),
        _full((HIDDEN, HIDDEN)),
        _full((HIDDEN, HIDDEN)),
        _full((1, HIDDEN)),
        _full((1, HIDDEN)),
        _full((HIDDEN, HIDDEN)),
        _full((1, HIDDEN)),
    ],
    out_specs=pl.BlockSpec((BE, HIDDEN), lambda i: (i, 0)),
    out_shape=jax.ShapeDtypeStruct((EH, HIDDEN), _F32),
)


RB = 2000


def _update_body(x_ref, a0_ref, a1_ref, W1x_ref, W1a_ref,
                 b1_ref, g_ref, bln_ref, W2_ref, b2_ref, out_ref):
    xb = x_ref[...]
    agg = a0_ref[...] + a1_ref[...]
    u = jnp.dot(xb, W1x_ref[...], preferred_element_type=_F32,
                precision=lax.Precision.HIGHEST)
    u += jnp.dot(agg, W1a_ref[...], preferred_element_type=_F32,
                 precision=lax.Precision.HIGHEST)
    u += b1_ref[...]
    mu = jnp.mean(u, axis=-1, keepdims=True)
    var = jnp.mean((u - mu) * (u - mu), axis=-1, keepdims=True)
    un = (u - mu) * lax.rsqrt(var + LN_EPS) * g_ref[...] + bln_ref[...]
    h = un * jax.nn.sigmoid(un)
    out_ref[...] = (jnp.dot(h, W2_ref[...], preferred_element_type=_F32,
                            precision=lax.Precision.HIGHEST)
                    + b2_ref[...] + xb)


_update = pl.pallas_call(
    _update_body,
    grid=(N_NODES // RB,),
    in_specs=[
        pl.BlockSpec((RB, HIDDEN), lambda i: (i, 0)),
        pl.BlockSpec((RB, HIDDEN), lambda i: (i, 0)),
        pl.BlockSpec((RB, HIDDEN), lambda i: (i, 0)),
        _full((HIDDEN, HIDDEN)),
        _full((HIDDEN, HIDDEN)),
        _full((1, HIDDEN)),
        _full((1, HIDDEN)),
        _full((1, HIDDEN)),
        _full((HIDDEN, HIDDEN)),
        _full((1, HIDDEN)),
    ],
    out_specs=pl.BlockSpec((RB, HIDDEN), lambda i: (i, 0)),
    out_shape=jax.ShapeDtypeStruct((N_NODES, HIDDEN), _F32),
)


def kernel(x, edge_index, edge_weight, W1m, b1m, W2m, b2m, W1u, b1u,
           ln_g, ln_b, W2u, b2u):
    src = edge_index[0].astype(jnp.int32)
    dst = edge_index[1].astype(jnp.int32)

    Ws = W1m[:HIDDEN].astype(jnp.bfloat16)
    Wd = W1m[HIDDEN:2 * HIDDEN].astype(jnp.bfloat16)
    wrow = W1m[2 * HIDDEN:].reshape(1, -1)
    b1 = b1m.reshape(1, -1)
    W2 = W2m.astype(jnp.bfloat16)
    b2 = b2m.reshape(1, -1)
    zeros = jnp.zeros((N_NODES, HIDDEN), _F32)
    w2d = edge_weight.reshape(-1, 1)

    src_g, dst_g = _make_gather(0)(x, src, dst)
    msg = _edge_mlp(src_g, dst_g, w2d, Ws, Wd, wrow, b1, W2, b2)
    parts = _make_scatter(0)(msg, dst, zeros)

    out = _update(
        x, parts[0], parts[1],
        W1u[:HIDDEN], W1u[HIDDEN:], b1u.reshape(1, -1),
        ln_g.reshape(1, -1), ln_b.reshape(1, -1), W2u, b2u.reshape(1, -1),
    )
    return out


# speedup vs baseline: 1.3568x; 1.0451x over previous
import functools

import jax
import jax.numpy as jnp
from jax import lax
from jax.experimental import pallas as pl
from jax.experimental.pallas import tpu as pltpu
from jax.experimental.pallas import tpu_sc as plsc

N_NODES = 10000
HIDDEN = 128
N_EDGES = 320000
LN_EPS = 1e-5

NC = 2
NS = 16
NW = NC * NS
EH = N_EDGES
EPW = EH // NW
GCHUNK = 80
GRING = 2
NCHG = EPW // GCHUNK
SCHUNK = 80
SRING = 4
NCHS = EPW // SCHUNK
NPS = 632
NPS_LAST = N_NODES - NPS * (NS - 1)

_F32 = jnp.float32


def _gather_body(ebase, x_hbm, src_hbm, dst_hbm, srcg_hbm, dstg_hbm,
                 xs, idx_s, idx_d, rows_s, rows_d, sem_ix, sem_g,
                 sem_ws, sem_wd):
    c = lax.axis_index("c")
    s = lax.axis_index("s")
    wid = s * NC + c
    base_out = wid * EPW
    base_in = ebase + base_out

    row0 = pl.multiple_of(s * NPS, 8)

    @pl.when(s < NS - 1)
    def _():
        pltpu.sync_copy(x_hbm.at[pl.ds(row0, NPS)], xs.at[pl.ds(row0, NPS)])

    @pl.when(s == NS - 1)
    def _():
        pltpu.sync_copy(x_hbm.at[pl.ds(NPS * (NS - 1), NPS_LAST)],
                        xs.at[pl.ds(NPS * (NS - 1), NPS_LAST)])

    plsc.subcore_barrier()

    def issue_idx(i, b):
        off = pl.multiple_of(base_in + i * GCHUNK, 8)
        pltpu.async_copy(src_hbm.at[pl.ds(off, GCHUNK)], idx_s[b], sem_ix[b])
        pltpu.async_copy(dst_hbm.at[pl.ds(off, GCHUNK)], idx_d[b], sem_ix[b])

    def wait_idx(b):
        pltpu.make_async_copy(src_hbm.at[pl.ds(0, GCHUNK)], idx_s[b],
                              sem_ix[b]).wait()
        pltpu.make_async_copy(dst_hbm.at[pl.ds(0, GCHUNK)], idx_d[b],
                              sem_ix[b]).wait()

    def wait_writes(b):
        pltpu.make_async_copy(
            rows_s[b], srcg_hbm.at[pl.ds(0, GCHUNK)], sem_ws[b]).wait()
        pltpu.make_async_copy(
            rows_d[b], dstg_hbm.at[pl.ds(0, GCHUNK)], sem_wd[b]).wait()

    def step(i, b, wait_w, last):
        off = pl.multiple_of(base_out + i * GCHUNK, 8)
        if wait_w:
            wait_writes(b)
        wait_idx(b)
        cp_s = pltpu.async_copy(xs.at[idx_s[b]], rows_s[b], sem_g)
        cp_d = pltpu.async_copy(xs.at[idx_d[b]], rows_d[b], sem_g)
        cp_s.wait()
        cp_d.wait()
        if not last:
            if isinstance(i, int):
                if i + GRING < NCHG:
                    issue_idx(i + GRING, b)
            else:
                @pl.when(i + GRING < NCHG)
                def _():
                    issue_idx(i + GRING, b)
        pltpu.async_copy(rows_s[b], srcg_hbm.at[pl.ds(off, GCHUNK)],
                         sem_ws[b])
        pltpu.async_copy(rows_d[b], dstg_hbm.at[pl.ds(off, GCHUNK)],
                         sem_wd[b])

    for b in range(GRING):
        issue_idx(b, b)

    for b in range(GRING):
        step(b, b, wait_w=False, last=False)

    @pl.loop(1, NCHG // GRING)
    def _pair(j):
        for b in range(GRING):
            step(j * GRING + b, b, wait_w=True, last=False)

    for t in range(NCHG - (NCHG // GRING) * GRING):
        i = (NCHG // GRING) * GRING + t
        step(i, i % GRING, wait_w=True, last=True)

    for b in range(GRING):
        wait_writes(b)


@functools.cache
def _make_gather(ebase):
    return pl.kernel(
        functools.partial(_gather_body, ebase),
        out_type=(
            jax.ShapeDtypeStruct((EH, HIDDEN), _F32),
            jax.ShapeDtypeStruct((EH, HIDDEN), _F32),
        ),
        mesh=plsc.VectorSubcoreMesh(core_axis_name="c", subcore_axis_name="s"),
        scratch_types=[
            pltpu.VMEM_SHARED((N_NODES, HIDDEN), _F32),
            [pltpu.VMEM((GCHUNK,), jnp.int32) for _ in range(GRING)],
            [pltpu.VMEM((GCHUNK,), jnp.int32) for _ in range(GRING)],
            [pltpu.VMEM((GCHUNK, HIDDEN), _F32) for _ in range(GRING)],
            [pltpu.VMEM((GCHUNK, HIDDEN), _F32) for _ in range(GRING)],
            [pltpu.SemaphoreType.DMA for _ in range(GRING)],
            pltpu.SemaphoreType.DMA,
            [pltpu.SemaphoreType.DMA for _ in range(GRING)],
            [pltpu.SemaphoreType.DMA for _ in range(GRING)],
        ],
    )


def _scatter_body(ebase, msg_hbm, dsti_hbm, zeros_hbm, out_hbm, idx_v, msg_v,
                  sem_ld, acc):
    c = lax.axis_index("c")
    s = lax.axis_index("s")
    wid = s * NC + c
    base_msg = wid * EPW
    base_in = ebase + base_msg

    row0 = pl.multiple_of(s * NPS, 8)

    @pl.when(s < NS - 1)
    def _():
        pltpu.sync_copy(zeros_hbm.at[pl.ds(row0, NPS)],
                        acc.at[pl.ds(row0, NPS)])

    @pl.when(s == NS - 1)
    def _():
        pltpu.sync_copy(zeros_hbm.at[pl.ds(NPS * (NS - 1), NPS_LAST)],
                        acc.at[pl.ds(NPS * (NS - 1), NPS_LAST)])

    plsc.subcore_barrier()

    def issue_load(i, b):
        offi = pl.multiple_of(base_in + i * SCHUNK, 8)
        offm = pl.multiple_of(base_msg + i * SCHUNK, 8)
        pltpu.async_copy(dsti_hbm.at[pl.ds(offi, SCHUNK)], idx_v[b],
                         sem_ld[b])
        pltpu.async_copy(msg_hbm.at[pl.ds(offm, SCHUNK)], msg_v[b],
                         sem_ld[b])

    def wait_load(b):
        pltpu.make_async_copy(dsti_hbm.at[pl.ds(0, SCHUNK)], idx_v[b],
                              sem_ld[b]).wait()
        pltpu.make_async_copy(msg_hbm.at[pl.ds(0, SCHUNK)], msg_v[b],
                              sem_ld[b]).wait()

    def sstep(i, b, last):
        wait_load(b)
        pltpu.sync_copy(msg_v[b], acc.at[idx_v[b]], add=True)
        if not last:
            if isinstance(i, int):
                if i + SRING < NCHS:
                    issue_load(i + SRING, b)
            else:
                @pl.when(i + SRING < NCHS)
                def _():
                    issue_load(i + SRING, b)

    for b in range(SRING):
        issue_load(b, b)
    for b in range(SRING):
        sstep(b, b, last=False)

    @pl.loop(1, NCHS // SRING)
    def _ring(j):
        for b in range(SRING):
            sstep(j * SRING + b, b, last=False)

    for t in range(NCHS - (NCHS // SRING) * SRING):
        i = (NCHS // SRING) * SRING + t
        sstep(i, i % SRING, last=True)

    plsc.subcore_barrier()

    @pl.when(s < NS - 1)
    def _():
        pltpu.sync_copy(acc.at[pl.ds(row0, NPS)],
                        out_hbm.at[c].at[pl.ds(row0, NPS)])

    @pl.when(s == NS - 1)
    def _():
        pltpu.sync_copy(acc.at[pl.ds(NPS * (NS - 1), NPS_LAST)],
                        out_hbm.at[c].at[pl.ds(NPS * (NS - 1), NPS_LAST)])


@functools.cache
def _make_scatter(ebase):
    return pl.kernel(
        functools.partial(_scatter_body, ebase),
        out_type=jax.ShapeDtypeStruct((NC, N_NODES, HIDDEN), _F32),
        mesh=plsc.VectorSubcoreMesh(core_axis_name="c", subcore_axis_name="s"),
        scratch_types=[
            [pltpu.VMEM((SCHUNK,), jnp.int32) for _ in range(SRING)],
            [pltpu.VMEM((SCHUNK, HIDDEN), _F32) for _ in range(SRING)],
            [pltpu.SemaphoreType.DMA for _ in range(SRING)],
            pltpu.VMEM_SHARED((N_NODES, HIDDEN), _F32),
        ],
    )


BE = 8000


def _edge_mlp_body(src_ref, dst_ref, w_ref, Ws_ref, Wd_ref, wrow_ref,
                   b1_ref, W2_ref, b2_ref, out_ref):
    w = w_ref[...]
    z = jnp.dot(src_ref[...].astype(jnp.bfloat16), Ws_ref[...],
                preferred_element_type=_F32)
    z += jnp.dot(dst_ref[...].astype(jnp.bfloat16), Wd_ref[...],
                 preferred_element_type=_F32)
    z += w * wrow_ref[...] + b1_ref[...]
    h = z * jax.nn.sigmoid(z)
    m = jnp.dot(h.astype(jnp.bfloat16), W2_ref[...],
                preferred_element_type=_F32) + b2_ref[...]
    out_ref[...] = m * w


def _full(shape):
    return pl.BlockSpec(shape, lambda i: (0, 0))


_edge_mlp = pl.pallas_call(
    _edge_mlp_body,
    grid=(EH // BE,),
    in_specs=[
        pl.BlockSpec((BE, HIDDEN), lambda i: (i, 0)),
        pl.BlockSpec((BE, HIDDEN), lambda i: (i, 0)),
        pl.BlockSpec((BE, 1), lambda i: (i, 0)),
        _full((HIDDEN, HIDDEN)),
        _full((HIDDEN, HIDDEN)),
        _full((1, HIDDEN)),
        _full((1, HIDDEN)),
        _full((HIDDEN, HIDDEN)),
        _full((1, HIDDEN)),
    ],
    out_specs=pl.BlockSpec((BE, HIDDEN), lambda i: (i, 0)),
    out_shape=jax.ShapeDtypeStruct((EH, HIDDEN), _F32),
)


RB = 2000


def _update_body(x_ref, a0_ref, a1_ref, W1x_ref, W1a_ref,
                 b1_ref, g_ref, bln_ref, W2_ref, b2_ref, out_ref):
    xb = x_ref[...]
    agg = a0_ref[...] + a1_ref[...]
    u = jnp.dot(xb.astype(jnp.bfloat16), W1x_ref[...].astype(jnp.bfloat16),
                preferred_element_type=_F32)
    u += jnp.dot(agg.astype(jnp.bfloat16),
                 W1a_ref[...].astype(jnp.bfloat16),
                 preferred_element_type=_F32)
    u += b1_ref[...]
    mu = jnp.mean(u, axis=-1, keepdims=True)
    var = jnp.mean((u - mu) * (u - mu), axis=-1, keepdims=True)
    un = (u - mu) * lax.rsqrt(var + LN_EPS) * g_ref[...] + bln_ref[...]
    h = un * jax.nn.sigmoid(un)
    out_ref[...] = (jnp.dot(h.astype(jnp.bfloat16),
                            W2_ref[...].astype(jnp.bfloat16),
                            preferred_element_type=_F32)
                    + b2_ref[...] + xb)


_update = pl.pallas_call(
    _update_body,
    grid=(N_NODES // RB,),
    in_specs=[
        pl.BlockSpec((RB, HIDDEN), lambda i: (i, 0)),
        pl.BlockSpec((RB, HIDDEN), lambda i: (i, 0)),
        pl.BlockSpec((RB, HIDDEN), lambda i: (i, 0)),
        _full((HIDDEN, HIDDEN)),
        _full((HIDDEN, HIDDEN)),
        _full((1, HIDDEN)),
        _full((1, HIDDEN)),
        _full((1, HIDDEN)),
        _full((HIDDEN, HIDDEN)),
        _full((1, HIDDEN)),
    ],
    out_specs=pl.BlockSpec((RB, HIDDEN), lambda i: (i, 0)),
    out_shape=jax.ShapeDtypeStruct((N_NODES, HIDDEN), _F32),
)


def kernel(x, edge_index, edge_weight, W1m, b1m, W2m, b2m, W1u, b1u,
           ln_g, ln_b, W2u, b2u):
    src = edge_index[0].astype(jnp.int32)
    dst = edge_index[1].astype(jnp.int32)

    Ws = W1m[:HIDDEN].astype(jnp.bfloat16)
    Wd = W1m[HIDDEN:2 * HIDDEN].astype(jnp.bfloat16)
    wrow = W1m[2 * HIDDEN:].reshape(1, -1)
    b1 = b1m.reshape(1, -1)
    W2 = W2m.astype(jnp.bfloat16)
    b2 = b2m.reshape(1, -1)
    zeros = jnp.zeros((N_NODES, HIDDEN), _F32)
    w2d = edge_weight.reshape(-1, 1)

    src_g, dst_g = _make_gather(0)(x, src, dst)
    msg = _edge_mlp(src_g, dst_g, w2d, Ws, Wd, wrow, b1, W2, b2)
    parts = _make_scatter(0)(msg, dst, zeros)

    out = _update(
        x, parts[0], parts[1],
        W1u[:HIDDEN], W1u[HIDDEN:], b1u.reshape(1, -1),
        ln_g.reshape(1, -1), ln_b.reshape(1, -1), W2u, b2u.reshape(1, -1),
    )
    return out
